# Initial kernel scaffold; baseline (speedup 1.0000x reference)
import jax
import jax.numpy as jnp
from jax.experimental import pallas as pl


def kernel(hidden_states, W_gate, e_bias, W_gate_up, W_down, Ws_gate_up, Ws_down):
    raise NotImplementedError("write your pallas kernel here")


# speedup vs baseline: 2.7962x; 2.7962x over previous
import functools
import jax
import jax.numpy as jnp
from jax import lax
from jax.experimental import pallas as pl
from jax.experimental.pallas import tpu as pltpu

T = 2048
D = 1024
E = 8
DFF = 512
NG = 4
TG = 2
K = 2
NS = 2
RSF = 2.5

NEG = jnp.finfo(jnp.float32).min


def _first_k_mask(vals, k, triu):
    n = vals.shape[1]
    rem = vals
    sel = jnp.zeros_like(vals, dtype=jnp.bool_)
    for _ in range(k):
        m = jnp.max(rem, axis=1, keepdims=True)
        eq = rem == m
        cnt = lax.dot_general(
            eq.astype(jnp.float32), triu,
            (((1,), (0,)), ((), ())),
            precision=lax.Precision.HIGHEST,
        )
        first = jnp.logical_and(eq, cnt == 1.0)
        sel = jnp.logical_or(sel, first)
        rem = jnp.where(first, NEG, rem)
    return sel


def _routing(x, wg, eb):
    logits = lax.dot_general(
        x, wg, (((1,), (0,)), ((), ())), precision=lax.Precision.DEFAULT)
    scores = jax.nn.sigmoid(logits)
    sfc = scores + eb

    r8 = lax.broadcasted_iota(jnp.int32, (E, NG), 0)
    c8 = lax.broadcasted_iota(jnp.int32, (E, NG), 1)
    G = (r8 // (E // NG) == c8).astype(jnp.float32)
    gsum = lax.dot_general(
        sfc, G, (((1,), (0,)), ((), ())), precision=lax.Precision.HIGHEST)

    rg = lax.broadcasted_iota(jnp.int32, (NG, NG), 0)
    cg = lax.broadcasted_iota(jnp.int32, (NG, NG), 1)
    triu_g = (rg <= cg).astype(jnp.float32)
    gmask = _first_k_mask(gsum, TG, triu_g)

    smask = lax.dot_general(
        gmask.astype(jnp.float32), G.T, (((1,), (0,)), ((), ())),
        precision=lax.Precision.HIGHEST) > 0.5
    masked = jnp.where(smask, sfc, NEG)

    re_ = lax.broadcasted_iota(jnp.int32, (E, E), 0)
    ce_ = lax.broadcasted_iota(jnp.int32, (E, E), 1)
    triu_e = (re_ <= ce_).astype(jnp.float32)
    sel = _first_k_mask(masked, K, triu_e)

    w = jnp.where(sel, scores, 0.0)
    wsum = jnp.sum(w, axis=1, keepdims=True) + 1e-20
    return w / wsum * RSF


TB = 1024


def _moe_body(x_ref, wg_ref, eb_ref, wgur_ref, wsg_ref, wsu_ref,
              wdr_ref, wds_ref, out_ref, comb_ref):
    e = pl.program_id(1)

    @pl.when(e == 0)
    def _():
        comb_ref[...] = _routing(x_ref[...], wg_ref[...], eb_ref[...])
        out_ref[...] = jnp.zeros_like(out_ref)

    x = x_ref[...]

    @pl.when(e < E)
    def _():
        gu = lax.dot_general(
            x, wgur_ref[0], (((1,), (0,)), ((), ())),
            precision=lax.Precision.DEFAULT)
        g = gu[:, :DFF]
        u = gu[:, DFF:]
        lane = lax.broadcasted_iota(jnp.int32, (TB, E), 1)
        wsel = jnp.sum(jnp.where(lane == e, comb_ref[...], 0.0),
                       axis=1, keepdims=True)
        h = g * jax.nn.sigmoid(g) * u * wsel
        out_ref[...] += lax.dot_general(
            h, wdr_ref[0], (((1,), (0,)), ((), ())),
            precision=lax.Precision.DEFAULT)

    @pl.when(e >= E)
    def _():
        g = lax.dot_general(
            x, wsg_ref[...], (((1,), (0,)), ((), ())),
            precision=lax.Precision.DEFAULT)
        u = lax.dot_general(
            x, wsu_ref[...], (((1,), (0,)), ((), ())),
            precision=lax.Precision.DEFAULT)
        h = g * jax.nn.sigmoid(g) * u
        out_ref[...] += lax.dot_general(
            h, wds_ref[0], (((1,), (0,)), ((), ())),
            precision=lax.Precision.DEFAULT)


@jax.jit
def _moe(hidden_states, W_gate, e_bias2, W_gate_up, W_down, Ws_gate_up,
         Ws_down3):
    grid = (T // TB, E + NS)
    clamp_r = lambda t, e: jnp.minimum(e, E - 1)
    clamp_s = lambda t, e: jnp.clip(e - E, 0, NS - 1)
    return pl.pallas_call(
        _moe_body,
        grid=grid,
        in_specs=[
            pl.BlockSpec((TB, D), lambda t, e: (t, 0)),
            pl.BlockSpec((D, E), lambda t, e: (0, 0)),
            pl.BlockSpec((1, E), lambda t, e: (0, 0)),
            pl.BlockSpec((1, D, 2 * DFF),
                         lambda t, e: (clamp_r(t, e), 0, 0)),
            pl.BlockSpec((D, DFF),
                         lambda t, e: (0, clamp_s(t, e))),
            pl.BlockSpec((D, DFF),
                         lambda t, e: (0, clamp_s(t, e) + NS)),
            pl.BlockSpec((1, DFF, D),
                         lambda t, e: (clamp_r(t, e), 0, 0)),
            pl.BlockSpec((1, DFF, D),
                         lambda t, e: (clamp_s(t, e), 0, 0)),
        ],
        out_specs=pl.BlockSpec((TB, D), lambda t, e: (t, 0)),
        out_shape=jax.ShapeDtypeStruct((T, D), jnp.float32),
        scratch_shapes=[pltpu.VMEM((TB, E), jnp.float32)],
        compiler_params=pltpu.CompilerParams(
            dimension_semantics=("arbitrary", "arbitrary")),
    )(hidden_states, W_gate, e_bias2, W_gate_up, Ws_gate_up, Ws_gate_up,
      W_down, Ws_down3)


def kernel(hidden_states, W_gate, e_bias, W_gate_up, W_down, Ws_gate_up,
           Ws_down):
    e_bias2 = e_bias.reshape(1, E)
    Ws_down3 = Ws_down.reshape(NS, DFF, D)
    return _moe(hidden_states, W_gate, e_bias2, W_gate_up, W_down,
                Ws_gate_up, Ws_down3)
